# baseline (speedup 1.0000x reference)
import functools

import jax
import jax.numpy as jnp
import numpy as np
from jax import lax
from jax.experimental import pallas as pl
from jax.experimental.pallas import tpu as pltpu
from jax.experimental.pallas import tpu_sc as plsc

N_PAD = 10240
N_SC = 10112
D = 128
DH = D // 2
E_TOTAL = 320000
E_PAD = 327680
CHUNK = 128
ROWS_PER_TILE = E_PAD // 32 // CHUNK
NODES_PER_TILE = N_SC // 16
NBUF = 2
DEN_ROWS = 160


def _prologue_body(x_ref, w_ref, a1_ref, a2_ref, ea_ref, a3_ref,
                   hlo_ref, hhi_ref, s1_ref, s2_ref, e3_ref):
    h = jnp.dot(x_ref[...], w_ref[...], preferred_element_type=jnp.float32)
    hlo_ref[...] = h[:, :DH].astype(jnp.bfloat16)
    hhi_ref[...] = h[:, DH:].astype(jnp.bfloat16)
    s1_ref[...] = jnp.sum(h * a1_ref[...][None, :], axis=1)
    s2_ref[...] = jnp.sum(h * a2_ref[...][None, :], axis=1)
    e3 = jnp.sum(ea_ref[...] * a3_ref[...][None, :], axis=1)
    e3_ref[...] = e3.reshape(e3_ref.shape)


def _prologue(Xp, W, a1, a2, edge_attr, a3):
    grid = 20
    nb = N_PAD // grid
    eb = E_PAD // grid
    de = edge_attr.shape[1]
    return pl.pallas_call(
        _prologue_body,
        grid=(grid,),
        in_specs=[
            pl.BlockSpec((nb, D), lambda i: (i, 0)),
            pl.BlockSpec((D, D), lambda i: (0, 0)),
            pl.BlockSpec((D,), lambda i: (0,)),
            pl.BlockSpec((D,), lambda i: (0,)),
            pl.BlockSpec((eb, de), lambda i: (i, 0)),
            pl.BlockSpec((de,), lambda i: (0,)),
        ],
        out_specs=[
            pl.BlockSpec((nb, DH), lambda i: (i, 0)),
            pl.BlockSpec((nb, DH), lambda i: (i, 0)),
            pl.BlockSpec((nb,), lambda i: (i,)),
            pl.BlockSpec((nb,), lambda i: (i,)),
            pl.BlockSpec((eb // D, D), lambda i: (i, 0)),
        ],
        out_shape=[
            jax.ShapeDtypeStruct((N_PAD, DH), jnp.bfloat16),
            jax.ShapeDtypeStruct((N_PAD, DH), jnp.bfloat16),
            jax.ShapeDtypeStruct((N_PAD,), jnp.float32),
            jax.ShapeDtypeStruct((N_PAD,), jnp.float32),
            jax.ShapeDtypeStruct((E_PAD // D, D), jnp.float32),
        ],
    )(Xp, W, a1, a2, edge_attr, a3)


def _sc_body(hlo_hbm, hhi_hbm, src_hbm, tgt_hbm, e3_hbm, s1_hbm, s2_hbm,
             hp_hbm,
             s1_v, s2_v, src_v, tgt_v, e3_v, den_v, rows_bufs, half_bufs,
             wv_v, red_v, acc_v, denf_sh, hps_sh, gsems, ssems):
    c = lax.axis_index("c")
    s = lax.axis_index("s")
    zero16 = jnp.zeros((16,), jnp.float32)

    pltpu.sync_copy(s1_hbm.at[pl.ds(0, N_SC)], s1_v)
    pltpu.sync_copy(s2_hbm.at[pl.ds(0, N_SC)], s2_v)

    def _zero_den(i, carry):
        for q in range(4):
            den_v[i, pl.ds(q * 16, 16)] = zero16
        return carry
    lax.fori_loop(0, DEN_ROWS, _zero_den, 0)

    def _score16(j, k):
        sl = pl.ds(k * 16, 16)
        s16 = src_v[j, sl]
        t16 = tgt_v[j, sl]
        sc = (plsc.load_gather(s1_v, [s16]) +
              plsc.load_gather(s2_v, [t16]) + e3_v[j, sl])
        sc = jnp.where(sc >= 0, sc, sc * jnp.float32(0.01))
        return s16, jnp.exp(sc)

    for half in range(2):
        r0 = s * (2 * ROWS_PER_TILE) + half * ROWS_PER_TILE
        pltpu.sync_copy(src_hbm.at[pl.ds(r0, ROWS_PER_TILE)], src_v)
        pltpu.sync_copy(tgt_hbm.at[pl.ds(r0, ROWS_PER_TILE)], tgt_v)
        pltpu.sync_copy(e3_hbm.at[pl.ds(r0, ROWS_PER_TILE)], e3_v)

        def _p1_row(j, carry):
            for k in range(CHUNK // 16):
                s16, p16 = _score16(j, k)
                plsc.addupdate_scatter(
                    den_v,
                    [lax.shift_right_logical(s16, 6), s16 & 63], p16)
            return carry
        lax.fori_loop(0, ROWS_PER_TILE, _p1_row, 0)

    pltpu.sync_copy(den_v, hps_sh.at[pl.ds(s * DEN_ROWS, DEN_ROWS)])
    plsc.subcore_barrier()
    share = DEN_ROWS // 16
    for rnd in range(4):
        for k in range(4):
            pltpu.sync_copy(
                hps_sh.at[pl.ds((rnd * 4 + k) * DEN_ROWS + s * share, share)],
                red_v.at[k])

        def _red_row(i, carry):
            for q in range(4):
                sl = pl.ds(q * 16, 16)
                a = red_v[0, i, sl]
                for k in range(1, 4):
                    a = a + red_v[k, i, sl]
                if rnd == 0:
                    acc_v[i, sl] = a
                else:
                    acc_v[i, sl] = acc_v[i, sl] + a
            return carry
        lax.fori_loop(0, share, _red_row, 0)
    pltpu.sync_copy(acc_v, denf_sh.at[pl.ds(s * share, share)])
    plsc.subcore_barrier()
    pltpu.sync_copy(denf_sh, den_v)

    r0 = s * (2 * ROWS_PER_TILE) + c * ROWS_PER_TILE
    pltpu.sync_copy(src_hbm.at[pl.ds(r0, ROWS_PER_TILE)], src_v)
    pltpu.sync_copy(tgt_hbm.at[pl.ds(r0, ROWS_PER_TILE)], tgt_v)
    pltpu.sync_copy(e3_hbm.at[pl.ds(r0, ROWS_PER_TILE)], e3_v)

    def _weights(j):
        for k in range(CHUNK // 16):
            sl = pl.ds(k * 16, 16)
            s16, p16 = _score16(j, k)
            d16 = plsc.load_gather(
                den_v, [lax.shift_right_logical(s16, 6), s16 & 63])
            wv_v[sl] = p16 / (d16 + jnp.float32(1e-16))

    def _scale(bf_buf, f32_buf):
        himask = jnp.full((16,), -65536, jnp.int32)

        def _scale_e(e, carry2):
            w = wv_v[pl.ds(e, 16)][0]
            for g in range(DH // 32):
                xi = plsc.bitcast(bf_buf[e, pl.ds(g * 32, 32)], jnp.int32)
                ev = plsc.bitcast(lax.shift_left(xi, 16), jnp.float32)
                od = plsc.bitcast(xi & himask, jnp.float32)
                f32_buf[e, pl.ds(g * 32, 16)] = ev * w
                f32_buf[e, pl.ds(g * 32 + 16, 16)] = od * w
            return carry2
        lax.fori_loop(0, CHUNK, _scale_e, 0)

    for h_half, half in ((hlo_hbm, 0), (hhi_hbm, 1)):
        buf0 = half_bufs[0]

        def _zero_rows(e, carry):
            for q in range(DH // 16):
                buf0[e, pl.ds(q * 16, 16)] = zero16
            return carry
        lax.fori_loop(0, CHUNK, _zero_rows, 0)
        for r in range(NODES_PER_TILE // CHUNK):
            pltpu.sync_copy(
                buf0,
                hps_sh.at[pl.ds(s * NODES_PER_TILE + r * CHUNK, CHUNK)])
        rem = NODES_PER_TILE % CHUNK
        pltpu.sync_copy(
            buf0.at[pl.ds(0, rem)],
            hps_sh.at[pl.ds(s * NODES_PER_TILE
                            + (NODES_PER_TILE // CHUNK) * CHUNK, rem)])
        plsc.subcore_barrier()

        for b in range(NBUF):
            pltpu.async_copy(h_half.at[tgt_v.at[b]], rows_bufs[b],
                             gsems.at[b])

        def _p2_step(i, carry):
            for b in range(NBUF):
                j = i * NBUF + b
                _weights(j)
                pltpu.make_async_copy(h_half.at[tgt_v.at[j]], rows_bufs[b],
                                      gsems.at[b]).wait()
                @pl.when(j >= NBUF)
                def _():
                    pltpu.make_async_copy(half_bufs[b],
                                          hps_sh.at[src_v.at[j - NBUF]],
                                          ssems.at[b]).wait()
                _scale(rows_bufs[b], half_bufs[b])
                pltpu.async_copy(half_bufs[b], hps_sh.at[src_v.at[j]],
                                 ssems.at[b], add=True)
                nj = j + NBUF

                @pl.when(nj < ROWS_PER_TILE)
                def _():
                    pltpu.async_copy(h_half.at[tgt_v.at[nj]], rows_bufs[b],
                                     gsems.at[b])
            return carry
        lax.fori_loop(0, ROWS_PER_TILE // NBUF, _p2_step, 0)

        for b in range(NBUF):
            j = ROWS_PER_TILE - NBUF + b
            pltpu.make_async_copy(half_bufs[j % NBUF],
                                  hps_sh.at[src_v.at[j]],
                                  ssems.at[j % NBUF]).wait()

        plsc.subcore_barrier()
        pltpu.sync_copy(
            hps_sh.at[pl.ds(s * NODES_PER_TILE, NODES_PER_TILE)],
            hp_hbm.at[c, half, pl.ds(s * NODES_PER_TILE, NODES_PER_TILE)])
        plsc.subcore_barrier()


def _sc_main(hlo, hhi, src2, tgt2, e32, s1, s2):
    mesh = plsc.VectorSubcoreMesh(core_axis_name="c", subcore_axis_name="s")
    kfn = functools.partial(
        pl.kernel,
        mesh=mesh,
        compiler_params=pltpu.CompilerParams(use_tc_tiling_on_sc=False,
                                             needs_layout_passes=False),
        out_type=jax.ShapeDtypeStruct((2, 2, N_SC, DH), jnp.float32),
        scratch_types=[
            pltpu.VMEM((N_SC,), jnp.float32),
            pltpu.VMEM((N_SC,), jnp.float32),
            pltpu.VMEM((ROWS_PER_TILE, CHUNK), jnp.int32),
            pltpu.VMEM((ROWS_PER_TILE, CHUNK), jnp.int32),
            pltpu.VMEM((ROWS_PER_TILE, CHUNK), jnp.float32),
            pltpu.VMEM((DEN_ROWS, 64), jnp.float32),
            [pltpu.VMEM((CHUNK, DH), jnp.bfloat16)
             for _ in range(NBUF)],
            [pltpu.VMEM((CHUNK, DH), jnp.float32)
             for _ in range(NBUF)],
            pltpu.VMEM((CHUNK + 16,), jnp.float32),
            pltpu.VMEM((4, DEN_ROWS // 16, 64), jnp.float32),
            pltpu.VMEM((DEN_ROWS // 16, 64), jnp.float32),
            pltpu.VMEM_SHARED((DEN_ROWS, 64), jnp.float32),
            pltpu.VMEM_SHARED((N_SC, DH), jnp.float32),
            pltpu.SemaphoreType.DMA((NBUF,)),
            pltpu.SemaphoreType.DMA((NBUF,)),
        ],
    )(_sc_body)
    return kfn(hlo, hhi, src2, tgt2, e32, s1, s2)


def _epilogue_body(hp_ref, out_ref):
    lo = hp_ref[0, 0] + hp_ref[1, 0]
    hi = hp_ref[0, 1] + hp_ref[1, 1]
    x = jnp.concatenate([lo, hi], axis=1)
    out_ref[...] = jnp.where(x > 0, x, jnp.exp(x) - 1.0)


def _epilogue(hp2):
    grid = 16
    nb = N_SC // grid
    return pl.pallas_call(
        _epilogue_body,
        grid=(grid,),
        in_specs=[pl.BlockSpec((2, 2, nb, DH), lambda i: (0, 0, i, 0))],
        out_specs=pl.BlockSpec((nb, D), lambda i: (i, 0)),
        out_shape=jax.ShapeDtypeStruct((N_SC, D), jnp.float32),
    )(hp2)


def kernel(X, edge_index, edge_attr, W, a):
    n, d = X.shape
    src = edge_index[0].astype(jnp.int32)
    tgt = edge_index[1].astype(jnp.int32)
    a1 = a[:d, 0]
    a2 = a[d:2 * d, 0]
    a3 = a[2 * d:, 0]
    Xp = jnp.pad(X, ((0, N_PAD - n), (0, 0)))
    e = edge_index.shape[1]
    eap = jnp.pad(edge_attr, ((0, E_PAD - e), (0, 0)))
    hlo, hhi, s1, s2, e3 = _prologue(Xp, W, a1, a2, eap, a3)
    pad_idx = jnp.full((E_PAD - e,), N_SC - 1, jnp.int32)
    src2 = jnp.concatenate([src, pad_idx]).reshape(-1, CHUNK)
    tgt2 = jnp.concatenate([tgt, pad_idx]).reshape(-1, CHUNK)
    e32 = e3.reshape(-1, CHUNK)
    hp2 = _sc_main(hlo, hhi, src2, tgt2, e32, s1, s2)
    out = _epilogue(hp2)
    inv = np.empty((D,), np.int32)
    for hf in range(2):
        for g in range(DH // 32):
            for k in range(16):
                inv[hf * DH + 32 * g + 2 * k] = hf * DH + 32 * g + k
                inv[hf * DH + 32 * g + 2 * k + 1] = hf * DH + 32 * g + 16 + k
    return out[:n, inv]


# speedup vs baseline: 1.5747x; 1.5747x over previous
import functools

import jax
import jax.numpy as jnp
from jax import lax
from jax.experimental import pallas as pl
from jax.experimental.pallas import tpu as pltpu
from jax.experimental.pallas import tpu_sc as plsc

N_PAD = 10240
N_SC = 10112
D = 128
DH = D // 2
E_TOTAL = 320000
E_PAD = 327680
CHUNK = 64
ROWS_PER_TILE = E_PAD // 32 // CHUNK
NODES_PER_TILE = N_SC // 16
NBUF = 4
DEN_ROWS = 160


def _prologue_body(x_ref, w_ref, a1_ref, a2_ref, ea_ref, a3_ref,
                   hlo_ref, hhi_ref, s1_ref, s2_ref, e3_ref):
    h = jnp.dot(x_ref[...], w_ref[...], preferred_element_type=jnp.float32)
    hlo_ref[...] = h[:, :DH]
    hhi_ref[...] = h[:, DH:]
    s1_ref[...] = jnp.sum(h * a1_ref[...][None, :], axis=1)
    s2_ref[...] = jnp.sum(h * a2_ref[...][None, :], axis=1)
    e3 = jnp.sum(ea_ref[...] * a3_ref[...][None, :], axis=1)
    e3_ref[...] = e3.reshape(e3_ref.shape)


def _prologue(Xp, W, a1, a2, edge_attr, a3):
    grid = 20
    nb = N_PAD // grid
    eb = E_PAD // grid
    de = edge_attr.shape[1]
    return pl.pallas_call(
        _prologue_body,
        grid=(grid,),
        in_specs=[
            pl.BlockSpec((nb, D), lambda i: (i, 0)),
            pl.BlockSpec((D, D), lambda i: (0, 0)),
            pl.BlockSpec((D,), lambda i: (0,)),
            pl.BlockSpec((D,), lambda i: (0,)),
            pl.BlockSpec((eb, de), lambda i: (i, 0)),
            pl.BlockSpec((de,), lambda i: (0,)),
        ],
        out_specs=[
            pl.BlockSpec((nb, DH), lambda i: (i, 0)),
            pl.BlockSpec((nb, DH), lambda i: (i, 0)),
            pl.BlockSpec((nb,), lambda i: (i,)),
            pl.BlockSpec((nb,), lambda i: (i,)),
            pl.BlockSpec((eb // D, D), lambda i: (i, 0)),
        ],
        out_shape=[
            jax.ShapeDtypeStruct((N_PAD, DH), jnp.float32),
            jax.ShapeDtypeStruct((N_PAD, DH), jnp.float32),
            jax.ShapeDtypeStruct((N_PAD,), jnp.float32),
            jax.ShapeDtypeStruct((N_PAD,), jnp.float32),
            jax.ShapeDtypeStruct((E_PAD // D, D), jnp.float32),
        ],
    )(Xp, W, a1, a2, edge_attr, a3)


def _sc_body(hlo_hbm, hhi_hbm, src_hbm, tgt_hbm, e3_hbm, s1_hbm, s2_hbm,
             hp_hbm,
             s1_v, s2_v, src_v, tgt_v, e3_v, den_v, rows_bufs, wv_v,
             red_v, acc_v, denf_sh, hps_sh, gsems, ssems):
    c = lax.axis_index("c")
    s = lax.axis_index("s")
    zero16 = jnp.zeros((16,), jnp.float32)

    pltpu.sync_copy(s1_hbm.at[pl.ds(0, N_SC)], s1_v)
    pltpu.sync_copy(s2_hbm.at[pl.ds(0, N_SC)], s2_v)

    def _zero_den(i, carry):
        for q in range(4):
            den_v[i, pl.ds(q * 16, 16)] = zero16
        return carry
    lax.fori_loop(0, DEN_ROWS, _zero_den, 0)

    def _score16(j, k):
        sl = pl.ds(k * 16, 16)
        s16 = src_v[j, sl]
        t16 = tgt_v[j, sl]
        sc = (plsc.load_gather(s1_v, [s16]) +
              plsc.load_gather(s2_v, [t16]) + e3_v[j, sl])
        sc = jnp.where(sc >= 0, sc, sc * jnp.float32(0.01))
        return s16, jnp.exp(sc)

    for half in range(2):
        r0 = s * (2 * ROWS_PER_TILE) + half * ROWS_PER_TILE
        pltpu.sync_copy(src_hbm.at[pl.ds(r0, ROWS_PER_TILE)], src_v)
        pltpu.sync_copy(tgt_hbm.at[pl.ds(r0, ROWS_PER_TILE)], tgt_v)
        pltpu.sync_copy(e3_hbm.at[pl.ds(r0, ROWS_PER_TILE)], e3_v)

        def _p1_row(j, carry):
            for k in range(CHUNK // 16):
                s16, p16 = _score16(j, k)
                plsc.addupdate_scatter(
                    den_v,
                    [lax.shift_right_logical(s16, 6), s16 & 63], p16)
            return carry
        lax.fori_loop(0, ROWS_PER_TILE, _p1_row, 0)

    pltpu.sync_copy(den_v, hps_sh.at[pl.ds(s * DEN_ROWS, DEN_ROWS)])
    plsc.subcore_barrier()
    share = DEN_ROWS // 16
    for rnd in range(4):
        for k in range(4):
            pltpu.sync_copy(
                hps_sh.at[pl.ds((rnd * 4 + k) * DEN_ROWS + s * share, share)],
                red_v.at[k])

        def _red_row(i, carry):
            for q in range(4):
                sl = pl.ds(q * 16, 16)
                a = red_v[0, i, sl]
                for k in range(1, 4):
                    a = a + red_v[k, i, sl]
                if rnd == 0:
                    acc_v[i, sl] = a
                else:
                    acc_v[i, sl] = acc_v[i, sl] + a
            return carry
        lax.fori_loop(0, share, _red_row, 0)
    pltpu.sync_copy(acc_v, denf_sh.at[pl.ds(s * share, share)])
    plsc.subcore_barrier()
    pltpu.sync_copy(denf_sh, den_v)

    r0 = s * (2 * ROWS_PER_TILE) + c * ROWS_PER_TILE
    pltpu.sync_copy(src_hbm.at[pl.ds(r0, ROWS_PER_TILE)], src_v)
    pltpu.sync_copy(tgt_hbm.at[pl.ds(r0, ROWS_PER_TILE)], tgt_v)
    pltpu.sync_copy(e3_hbm.at[pl.ds(r0, ROWS_PER_TILE)], e3_v)

    def _weights(j):
        for k in range(CHUNK // 16):
            sl = pl.ds(k * 16, 16)
            s16, p16 = _score16(j, k)
            d16 = plsc.load_gather(
                den_v, [lax.shift_right_logical(s16, 6), s16 & 63])
            wv_v[sl] = p16 / (d16 + jnp.float32(1e-16))

    def _scale(buf):
        def _scale_e(e, carry2):
            w = wv_v[pl.ds(e, 16)][0]
            for q in range(DH // 16):
                ql = pl.ds(q * 16, 16)
                buf[e, ql] = buf[e, ql] * w
            return carry2
        lax.fori_loop(0, CHUNK, _scale_e, 0)

    for h_half, half in ((hlo_hbm, 0), (hhi_hbm, 1)):
        buf0 = rows_bufs[0]

        def _zero_rows(e, carry):
            for q in range(DH // 16):
                buf0[e, pl.ds(q * 16, 16)] = zero16
            return carry
        lax.fori_loop(0, CHUNK, _zero_rows, 0)
        for r in range(NODES_PER_TILE // CHUNK):
            pltpu.sync_copy(
                buf0,
                hps_sh.at[pl.ds(s * NODES_PER_TILE + r * CHUNK, CHUNK)])
        rem = NODES_PER_TILE % CHUNK
        pltpu.sync_copy(
            buf0.at[pl.ds(0, rem)],
            hps_sh.at[pl.ds(s * NODES_PER_TILE
                            + (NODES_PER_TILE // CHUNK) * CHUNK, rem)])
        plsc.subcore_barrier()

        for b in range(NBUF):
            pltpu.async_copy(h_half.at[tgt_v.at[b]], rows_bufs[b],
                             gsems.at[b])

        def _p2_step(i, carry):
            for b in range(NBUF):
                j = i * NBUF + b
                _weights(j)
                pltpu.make_async_copy(h_half.at[tgt_v.at[j]], rows_bufs[b],
                                      gsems.at[b]).wait()
                _scale(rows_bufs[b])
                pltpu.async_copy(rows_bufs[b], hps_sh.at[src_v.at[j]],
                                 ssems.at[b], add=True)
                pb = (b - 1) % NBUF
                pj = j - 1
                nj = pj + NBUF

                @pl.when((pj >= 0) & (nj < ROWS_PER_TILE))
                def _():
                    pltpu.make_async_copy(rows_bufs[pb],
                                          hps_sh.at[src_v.at[pj]],
                                          ssems.at[pb]).wait()
                    pltpu.async_copy(h_half.at[tgt_v.at[nj]], rows_bufs[pb],
                                     gsems.at[pb])
            return carry
        lax.fori_loop(0, ROWS_PER_TILE // NBUF, _p2_step, 0)

        for b in range(NBUF):
            j = ROWS_PER_TILE - NBUF + b
            pltpu.make_async_copy(rows_bufs[j % NBUF],
                                  hps_sh.at[src_v.at[j]],
                                  ssems.at[j % NBUF]).wait()

        plsc.subcore_barrier()
        pltpu.sync_copy(
            hps_sh.at[pl.ds(s * NODES_PER_TILE, NODES_PER_TILE)],
            hp_hbm.at[c, half, pl.ds(s * NODES_PER_TILE, NODES_PER_TILE)])
        plsc.subcore_barrier()


def _sc_main(hlo, hhi, src2, tgt2, e32, s1, s2):
    mesh = plsc.VectorSubcoreMesh(core_axis_name="c", subcore_axis_name="s")
    kfn = functools.partial(
        pl.kernel,
        mesh=mesh,
        compiler_params=pltpu.CompilerParams(use_tc_tiling_on_sc=False,
                                             needs_layout_passes=False),
        out_type=jax.ShapeDtypeStruct((2, 2, N_SC, DH), jnp.float32),
        scratch_types=[
            pltpu.VMEM((N_SC,), jnp.float32),
            pltpu.VMEM((N_SC,), jnp.float32),
            pltpu.VMEM((ROWS_PER_TILE, CHUNK), jnp.int32),
            pltpu.VMEM((ROWS_PER_TILE, CHUNK), jnp.int32),
            pltpu.VMEM((ROWS_PER_TILE, CHUNK), jnp.float32),
            pltpu.VMEM((DEN_ROWS, 64), jnp.float32),
            [pltpu.VMEM((CHUNK, DH), jnp.float32)
             for _ in range(NBUF)],
            pltpu.VMEM((CHUNK + 16,), jnp.float32),
            pltpu.VMEM((4, DEN_ROWS // 16, 64), jnp.float32),
            pltpu.VMEM((DEN_ROWS // 16, 64), jnp.float32),
            pltpu.VMEM_SHARED((DEN_ROWS, 64), jnp.float32),
            pltpu.VMEM_SHARED((N_SC, DH), jnp.float32),
            pltpu.SemaphoreType.DMA((NBUF,)),
            pltpu.SemaphoreType.DMA((NBUF,)),
        ],
    )(_sc_body)
    return kfn(hlo, hhi, src2, tgt2, e32, s1, s2)


def _epilogue_body(hp_ref, out_ref):
    lo = hp_ref[0, 0] + hp_ref[1, 0]
    hi = hp_ref[0, 1] + hp_ref[1, 1]
    x = jnp.concatenate([lo, hi], axis=1)
    out_ref[...] = jnp.where(x > 0, x, jnp.exp(x) - 1.0)


def _epilogue(hp2):
    grid = 16
    nb = N_SC // grid
    return pl.pallas_call(
        _epilogue_body,
        grid=(grid,),
        in_specs=[pl.BlockSpec((2, 2, nb, DH), lambda i: (0, 0, i, 0))],
        out_specs=pl.BlockSpec((nb, D), lambda i: (i, 0)),
        out_shape=jax.ShapeDtypeStruct((N_SC, D), jnp.float32),
    )(hp2)


def kernel(X, edge_index, edge_attr, W, a):
    n, d = X.shape
    src = edge_index[0].astype(jnp.int32)
    tgt = edge_index[1].astype(jnp.int32)
    a1 = a[:d, 0]
    a2 = a[d:2 * d, 0]
    a3 = a[2 * d:, 0]
    Xp = jnp.pad(X, ((0, N_PAD - n), (0, 0)))
    e = edge_index.shape[1]
    eap = jnp.pad(edge_attr, ((0, E_PAD - e), (0, 0)))
    hlo, hhi, s1, s2, e3 = _prologue(Xp, W, a1, a2, eap, a3)
    pad_idx = jnp.full((E_PAD - e,), N_SC - 1, jnp.int32)
    src2 = jnp.concatenate([src, pad_idx]).reshape(-1, CHUNK)
    tgt2 = jnp.concatenate([tgt, pad_idx]).reshape(-1, CHUNK)
    e32 = e3.reshape(-1, CHUNK)
    hp2 = _sc_main(hlo, hhi, src2, tgt2, e32, s1, s2)
    out = _epilogue(hp2)
    return out[:n]
